# baseline (speedup 1.0000x reference)
import functools

import jax
import jax.numpy as jnp
from jax import lax
from jax.experimental import pallas as pl
from jax.experimental.pallas import tpu as pltpu
from jax.experimental.pallas import tpu_sc as plsc

_NUM_MODES = 64
_BATCH = 16384
_COLS = 512

_KS = (0, 42, 0x1BD11BDA ^ 42)
_ROT = ((13, 15, 26, 6), (17, 29, 16, 24))

_NC, _NS, _L = 2, 16, 16
_NW = _NC * _NS
_BPW = _BATCH // _NW


def _threefry_bits(p):
    ks = tuple(jnp.uint32(k) for k in _KS)
    x0 = jnp.zeros_like(p) + ks[0]
    x1 = p + ks[1]
    for i in range(5):
        for r in _ROT[i % 2]:
            x0 = x0 + x1
            x1 = (x1 << jnp.uint32(r)) | (x1 >> jnp.uint32(32 - r))
            x1 = x0 ^ x1
        x0 = x0 + ks[(i + 1) % 3]
        x1 = x1 + ks[(i + 2) % 3] + jnp.uint32(i + 1)
    return x0 ^ x1


def _sample_body(zt_ref, w_ref, idx_ref):
    shp = (_NUM_MODES, _COLS)
    base = (pl.program_id(0) * _COLS).astype(jnp.uint32)
    lane = lax.broadcasted_iota(jnp.uint32, shp, 1) + base
    mode = lax.broadcasted_iota(jnp.uint32, shp, 0)
    p = lane * jnp.uint32(_NUM_MODES) + mode

    bits = _threefry_bits(p)
    fb = (bits >> jnp.uint32(9)) | jnp.uint32(0x3F800000)
    u = lax.bitcast_convert_type(fb, jnp.float32) - jnp.float32(1.0)
    tiny = jnp.float32(jnp.finfo(jnp.float32).tiny)
    unif = jnp.maximum(tiny, u * (jnp.float32(1.0) - tiny) + tiny)
    g = -jnp.log(-jnp.log(unif))

    def _b(x):
        return x.astype(jnp.bfloat16).astype(jnp.float32)

    logits = (_b(zt_ref[0:1, :]) * _b(w_ref[:, 0:1])
              + _b(zt_ref[1:2, :]) * _b(w_ref[:, 1:2]))
    val = g + logits

    m = jnp.max(val, axis=0, keepdims=True)
    modei = lax.broadcasted_iota(jnp.int32, shp, 0)
    cand = jnp.where(val == m, modei, jnp.int32(_NUM_MODES))
    idx_ref[...] = jnp.min(cand, axis=0, keepdims=True)


_SC_MESH = plsc.VectorSubcoreMesh(core_axis_name="c", subcore_axis_name="s")


@functools.partial(
    pl.kernel,
    mesh=_SC_MESH,
    out_type=jax.ShapeDtypeStruct((_BATCH * 2,), jnp.float32),
    scratch_types=[
        pltpu.VMEM((_BPW,), jnp.int32),
        pltpu.VMEM((_NUM_MODES * 2,), jnp.float32),
        pltpu.VMEM((_BPW * 2,), jnp.float32),
    ],
    compiler_params=pltpu.CompilerParams(needs_layout_passes=False),
)
def _gather_sc(idx_hbm, tab_hbm, out_hbm, idx_v, tab_v, out_v):
    wid = lax.axis_index("s") * _NC + lax.axis_index("c")
    base = wid * _BPW
    pltpu.sync_copy(idx_hbm.at[pl.ds(base, _BPW)], idx_v)
    pltpu.sync_copy(tab_hbm, tab_v)

    def body(j, carry):
        iv = idx_v[pl.ds(j * _L, _L)]
        b2 = iv * 2
        v0 = plsc.load_gather(tab_v, [b2])
        v1 = plsc.load_gather(tab_v, [b2 + 1])
        jj = (lax.iota(jnp.int32, _L) + j * _L) * 2
        plsc.store_scatter(out_v, [jj], v0)
        plsc.store_scatter(out_v, [jj + 1], v1)
        return carry

    lax.fori_loop(0, _BPW // _L, body, 0)
    pltpu.sync_copy(out_v, out_hbm.at[pl.ds(base * 2, _BPW * 2)])


@jax.jit
def _run(z2_onehot, W, embedding_table):
    zt = z2_onehot.T
    grid = (_BATCH // _COLS,)
    idx = pl.pallas_call(
        _sample_body,
        grid=grid,
        in_specs=[
            pl.BlockSpec((2, _COLS), lambda i: (0, i)),
            pl.BlockSpec((_NUM_MODES, 2), lambda i: (0, 0)),
        ],
        out_specs=pl.BlockSpec((1, _COLS), lambda i: (0, i)),
        out_shape=jax.ShapeDtypeStruct((1, _BATCH), jnp.int32),
    )(zt, W)
    out_flat = _gather_sc(idx.reshape(_BATCH), embedding_table.reshape(-1))
    return out_flat.reshape(_BATCH, 2)


def kernel(z2_onehot, W, embedding_table):
    return _run(z2_onehot, W, embedding_table)


# speedup vs baseline: 1.1146x; 1.1146x over previous
import functools

import jax
import jax.numpy as jnp
from jax import lax
from jax.experimental import pallas as pl
from jax.experimental.pallas import tpu as pltpu
from jax.experimental.pallas import tpu_sc as plsc

_NUM_MODES = 64
_BATCH = 16384
_COLS = 512

_KS = (0, 42, 0x1BD11BDA ^ 42)
_ROT = ((13, 15, 26, 6), (17, 29, 16, 24))

_NC, _NS, _L = 2, 16, 16
_NW = _NC * _NS
_BPW = _BATCH // _NW


def _threefry_bits(p):
    ks = tuple(jnp.uint32(k) for k in _KS)
    x0 = jnp.zeros_like(p) + ks[0]
    x1 = p + ks[1]
    for i in range(5):
        for r in _ROT[i % 2]:
            x0 = x0 + x1
            x1 = (x1 << jnp.uint32(r)) | (x1 >> jnp.uint32(32 - r))
            x1 = x0 ^ x1
        x0 = x0 + ks[(i + 1) % 3]
        x1 = x1 + ks[(i + 2) % 3] + jnp.uint32(i + 1)
    return x0 ^ x1


def _sample_body(zt_ref, w_ref, idx_ref):
    shp = (_NUM_MODES, _COLS)
    base = (pl.program_id(0) * _COLS).astype(jnp.uint32)
    lane = lax.broadcasted_iota(jnp.uint32, shp, 1) + base
    mode = lax.broadcasted_iota(jnp.uint32, shp, 0)
    p = lane * jnp.uint32(_NUM_MODES) + mode

    bits = _threefry_bits(p)
    fb = (bits >> jnp.uint32(9)) | jnp.uint32(0x3F800000)
    u = lax.bitcast_convert_type(fb, jnp.float32) - jnp.float32(1.0)
    tiny = jnp.float32(jnp.finfo(jnp.float32).tiny)
    unif = jnp.maximum(tiny, u * (jnp.float32(1.0) - tiny) + tiny)
    g = -jnp.log(-jnp.log(unif))

    def _b(x):
        return x.astype(jnp.bfloat16).astype(jnp.float32)

    logits = (_b(zt_ref[0:1, :]) * _b(w_ref[:, 0:1])
              + _b(zt_ref[1:2, :]) * _b(w_ref[:, 1:2]))
    val = g + logits

    m = jnp.max(val, axis=0, keepdims=True)
    modei = lax.broadcasted_iota(jnp.int32, shp, 0)
    cand = jnp.where(val == m, modei, jnp.int32(_NUM_MODES))
    idx_ref[...] = jnp.min(cand, axis=0, keepdims=True)


_SC_MESH = plsc.VectorSubcoreMesh(core_axis_name="c", subcore_axis_name="s")


@functools.partial(
    pl.kernel,
    mesh=_SC_MESH,
    out_type=jax.ShapeDtypeStruct((_BATCH, 2), jnp.float32),
    scratch_types=[
        pltpu.VMEM((_BPW,), jnp.int32),
        pltpu.VMEM((_NUM_MODES, 2), jnp.float32),
        pltpu.VMEM((_BPW, 2), jnp.float32),
    ],
    compiler_params=pltpu.CompilerParams(needs_layout_passes=False),
)
def _gather_sc(idx_hbm, tab_hbm, out_hbm, idx_v, tab_v, out_v):
    wid = lax.axis_index("s") * _NC + lax.axis_index("c")
    base = wid * _BPW
    pltpu.sync_copy(idx_hbm.at[0, pl.ds(base, _BPW)], idx_v)
    pltpu.sync_copy(tab_hbm, tab_v)

    def body(j, carry):
        iv = idx_v[pl.ds(j * _L, _L)]
        jr = lax.iota(jnp.int32, _L) + j * _L
        c0 = jnp.zeros((_L,), jnp.int32)
        c1 = c0 + 1
        v0 = plsc.load_gather(tab_v, [iv, c0])
        v1 = plsc.load_gather(tab_v, [iv, c1])
        plsc.store_scatter(out_v, [jr, c0], v0)
        plsc.store_scatter(out_v, [jr, c1], v1)
        return carry

    lax.fori_loop(0, _BPW // _L, body, 0)
    pltpu.sync_copy(out_v, out_hbm.at[pl.ds(base, _BPW), :])


@jax.jit
def _run(z2_onehot, W, embedding_table):
    zt = z2_onehot.T
    grid = (_BATCH // _COLS,)
    idx = pl.pallas_call(
        _sample_body,
        grid=grid,
        in_specs=[
            pl.BlockSpec((2, _COLS), lambda i: (0, i)),
            pl.BlockSpec((_NUM_MODES, 2), lambda i: (0, 0)),
        ],
        out_specs=pl.BlockSpec((1, _COLS), lambda i: (0, i)),
        out_shape=jax.ShapeDtypeStruct((1, _BATCH), jnp.int32),
    )(zt, W)
    return _gather_sc(idx, embedding_table)


def kernel(z2_onehot, W, embedding_table):
    return _run(z2_onehot, W, embedding_table)


# speedup vs baseline: 1.3752x; 1.2338x over previous
import functools

import jax
import jax.numpy as jnp
from jax import lax
from jax.experimental import pallas as pl

_NUM_MODES = 64
_BATCH = 16384
_COLS = 512

_KS = (0, 42, 0x1BD11BDA ^ 42)
_ROT = ((13, 15, 26, 6), (17, 29, 16, 24))


def _threefry_bits(p):
    ks = tuple(jnp.uint32(k) for k in _KS)
    x1 = p + ks[1]
    x0 = x1
    x1 = (x1 << jnp.uint32(13)) | (x1 >> jnp.uint32(19))
    x1 = x0 ^ x1
    for i in range(5):
        for r in _ROT[i % 2][1 if i == 0 else 0:]:
            x0 = x0 + x1
            x1 = (x1 << jnp.uint32(r)) | (x1 >> jnp.uint32(32 - r))
            x1 = x0 ^ x1
        x0 = x0 + ks[(i + 1) % 3]
        x1 = x1 + ks[(i + 2) % 3] + jnp.uint32(i + 1)
    return x0 ^ x1


def _sample_body(z_ref, w_ref, tab_ref, out_ref):
    shp = (_NUM_MODES, _COLS)
    base = (pl.program_id(0) * _COLS).astype(jnp.uint32)
    lane = lax.broadcasted_iota(jnp.uint32, shp, 1) + base
    mode = lax.broadcasted_iota(jnp.uint32, shp, 0)
    p = lane * jnp.uint32(_NUM_MODES) + mode

    bits = _threefry_bits(p)
    fb = (bits >> jnp.uint32(9)) | jnp.uint32(0x3F800000)
    u = lax.bitcast_convert_type(fb, jnp.float32) - jnp.float32(1.0)
    tiny = jnp.float32(jnp.finfo(jnp.float32).tiny)
    unif = jnp.maximum(tiny, u * (jnp.float32(1.0) - tiny) + tiny)
    g = -jnp.log(-jnp.log(unif))

    def _b(x):
        return x.astype(jnp.bfloat16).astype(jnp.float32)

    zt = z_ref[...].T
    logits = (_b(zt[0:1, :]) * _b(w_ref[:, 0:1])
              + _b(zt[1:2, :]) * _b(w_ref[:, 1:2]))
    val = g + logits

    m = jnp.max(val, axis=0, keepdims=True)
    modei = lax.broadcasted_iota(jnp.int32, shp, 0)
    cand = jnp.where(val == m, modei, jnp.int32(_NUM_MODES))
    idx = jnp.min(cand, axis=0, keepdims=True)

    onehot = ((val == m) & (modei == idx)).astype(jnp.float32)
    e0 = jnp.sum(onehot * tab_ref[:, 0:1], axis=0, keepdims=True)
    e1 = jnp.sum(onehot * tab_ref[:, 1:2], axis=0, keepdims=True)
    out_ref[...] = jnp.concatenate([e0, e1], axis=0).T


@jax.jit
def _run(z2_onehot, W, embedding_table):
    grid = (_BATCH // _COLS,)
    return pl.pallas_call(
        _sample_body,
        grid=grid,
        in_specs=[
            pl.BlockSpec((_COLS, 2), lambda i: (i, 0)),
            pl.BlockSpec((_NUM_MODES, 2), lambda i: (0, 0)),
            pl.BlockSpec((_NUM_MODES, 2), lambda i: (0, 0)),
        ],
        out_specs=pl.BlockSpec((_COLS, 2), lambda i: (i, 0)),
        out_shape=jax.ShapeDtypeStruct((_BATCH, 2), jnp.float32),
    )(z2_onehot, W, embedding_table)


def kernel(z2_onehot, W, embedding_table):
    return _run(z2_onehot, W, embedding_table)


# speedup vs baseline: 2.2750x; 1.6543x over previous
import jax
import jax.numpy as jnp
from jax import lax
from jax.experimental import pallas as pl

_NUM_MODES = 64
_BATCH = 16384
_COLS = 512

_KS = (0, 42, 0x1BD11BDA ^ 42)
_ROT = ((13, 15, 26, 6), (17, 29, 16, 24))


def _threefry_bits(p):
    ks = tuple(jnp.uint32(k) for k in _KS)
    x1 = p + ks[1]
    x0 = x1
    x1 = (x1 << jnp.uint32(13)) | (x1 >> jnp.uint32(19))
    x1 = x0 ^ x1
    for i in range(5):
        for r in _ROT[i % 2][1 if i == 0 else 0:]:
            x0 = x0 + x1
            x1 = (x1 << jnp.uint32(r)) | (x1 >> jnp.uint32(32 - r))
            x1 = x0 ^ x1
        x0 = x0 + ks[(i + 1) % 3]
        x1 = x1 + ks[(i + 2) % 3] + jnp.uint32(i + 1)
    return x0 ^ x1


def _sample_body(zt_ref, w_ref, tab_ref, out_ref):
    shp = (_NUM_MODES, _COLS)
    base = (pl.program_id(0) * _COLS).astype(jnp.uint32)
    lane = lax.broadcasted_iota(jnp.uint32, shp, 1) + base
    mode = lax.broadcasted_iota(jnp.uint32, shp, 0)
    p = lane * jnp.uint32(_NUM_MODES) + mode

    bits = _threefry_bits(p)
    fb = (bits >> jnp.uint32(9)) | jnp.uint32(0x3F800000)
    u = lax.bitcast_convert_type(fb, jnp.float32) - jnp.float32(1.0)
    tiny = jnp.float32(jnp.finfo(jnp.float32).tiny)
    unif = jnp.maximum(tiny, u * (jnp.float32(1.0) - tiny) + tiny)
    g = -jnp.log(-jnp.log(unif))

    def _b(x):
        return x.astype(jnp.bfloat16).astype(jnp.float32)

    logits = (_b(zt_ref[0:1, :]) * _b(w_ref[:, 0:1])
              + _b(zt_ref[1:2, :]) * _b(w_ref[:, 1:2]))
    val = g + logits

    m = jnp.max(val, axis=0, keepdims=True)
    modei = lax.broadcasted_iota(jnp.int32, shp, 0)
    cand = jnp.where(val == m, modei, jnp.int32(_NUM_MODES))
    idx = jnp.min(cand, axis=0, keepdims=True)

    onehot = ((val == m) & (modei == idx)).astype(jnp.float32)
    e0 = jnp.sum(onehot * tab_ref[:, 0:1], axis=0, keepdims=True)
    e1 = jnp.sum(onehot * tab_ref[:, 1:2], axis=0, keepdims=True)
    out_ref[...] = jnp.concatenate([e0, e1], axis=0)


@jax.jit
def _run(z2_onehot, W, embedding_table):
    zt = z2_onehot.T
    grid = (_BATCH // _COLS,)
    out = pl.pallas_call(
        _sample_body,
        grid=grid,
        in_specs=[
            pl.BlockSpec((2, _COLS), lambda i: (0, i)),
            pl.BlockSpec((_NUM_MODES, 2), lambda i: (0, 0)),
            pl.BlockSpec((_NUM_MODES, 2), lambda i: (0, 0)),
        ],
        out_specs=pl.BlockSpec((2, _COLS), lambda i: (0, i)),
        out_shape=jax.ShapeDtypeStruct((2, _BATCH), jnp.float32),
    )(zt, W, embedding_table)
    return out.T


def kernel(z2_onehot, W, embedding_table):
    return _run(z2_onehot, W, embedding_table)
